# Initial kernel scaffold; baseline (speedup 1.0000x reference)
import jax
import jax.numpy as jnp
from jax.experimental import pallas as pl


def kernel(x, Wq, Wk, Wv, Wout):
    raise NotImplementedError("write your pallas kernel here")


# speedup vs baseline: 8.4425x; 8.4425x over previous
import functools

import jax
import jax.numpy as jnp
from jax.experimental import pallas as pl
from jax.experimental.pallas import tpu as pltpu

B, N, T, F = 2, 207, 1024, 64
H = 8
PATCH = 8
TOPK = 4
FPH = F // H
TP = T // PATCH

def _split(a):
    hi = a.astype(jnp.bfloat16)
    lo = (a - hi.astype(jnp.float32)).astype(jnp.bfloat16)
    return hi, lo


def _dot(a, b):
    ah, al = _split(a)
    bh, bl = _split(b)

    def d(u, v):
        return jax.lax.dot(u, v, preferred_element_type=jnp.float32)

    return d(ah, bh) + d(ah, bl) + d(al, bh)


def _dot_bf16(a, b):
    return jax.lax.dot(a.astype(jnp.bfloat16), b.astype(jnp.bfloat16),
                       preferred_element_type=jnp.float32)


def _ac_kernel(x_ref, wq_ref, wk_ref, wv_ref, wsum_ref, c_ref, s_ref, o_ref):
    X = x_ref[0, 0]
    C = c_ref[...]
    S = s_ref[...]

    q = _dot_bf16(X, wq_ref[...])
    k = _dot_bf16(X, wk_ref[...])
    V2 = _dot_bf16(X, wv_ref[...])

    Qr = _dot(C, q)
    Qi = -_dot(S, q)
    Kr = _dot(C, k)
    Ki = -_dot(S, k)
    Pr = Qr * Kr + Qi * Ki
    Pi = Qi * Kr - Qr * Ki
    corr = (_dot(C, Pr) - _dot(S, Pi)) * (1.0 / TP)

    rowi = jax.lax.broadcasted_iota(jnp.int32, (TP, H), 0)
    work = corr
    ws = []
    ds = []
    for _ in range(TOPK):
        m = jnp.max(work, axis=0, keepdims=True)
        idx = jnp.min(jnp.where(work == m, rowi, TP), axis=0,
                      keepdims=True)
        ws.append(m)
        ds.append(idx)
        work = jnp.where(rowi == idx, -jnp.inf, work)
    mx = ws[0]
    es = [jnp.exp(w - mx) for w in ws]
    z = es[0] + es[1] + es[2] + es[3]
    sms = [e / z for e in es]

    s2 = jnp.zeros((TP, H), jnp.float32)
    for sm_i, d_i in zip(sms, ds):
        s2 = s2 + jnp.where(rowi == d_i, sm_i, 0.0)

    Sr = _dot(C, s2)
    Si = -_dot(S, s2)
    srB = jnp.broadcast_to(Sr[:, None, :], (TP, PATCH, H)).reshape(TP, F)
    siB = jnp.broadcast_to(Si[:, None, :], (TP, PATCH, H)).reshape(TP, F)
    VFr = _dot(C, V2)
    VFi = -_dot(S, V2)
    Pr2 = VFr * srB + VFi * siB
    Pi2 = VFi * srB - VFr * siB
    aggr = (_dot(C, Pr2) - _dot(S, Pi2)) * (1.0 / TP)
    aggm = aggr.reshape(TP, PATCH, H).sum(axis=2) * (1.0 / H)
    agg = aggm.reshape(1, T)
    agg = agg.astype(jnp.bfloat16).astype(jnp.float32)
    o_ref[0, 0] = wsum_ref[...] * agg


@functools.partial(jax.jit, static_argnames=("interpret",))
def kernel(x, Wq, Wk, Wv, Wout, interpret=False):
    xr = x.reshape(B, N, TP, PATCH * F)
    Wq2 = Wq.transpose(2, 1, 0).reshape(PATCH * F, H)
    Wk2 = Wk.transpose(2, 1, 0).reshape(PATCH * F, H)
    Wvblk = jnp.kron(jnp.eye(PATCH, dtype=jnp.float32), Wv.T)
    Wsum = (Wout.astype(jnp.bfloat16).astype(jnp.float32)
            .sum(axis=1).reshape(F, 1))
    idx = jnp.arange(TP, dtype=jnp.float32)
    ang = (2.0 * jnp.pi / TP) * (idx[:, None] * idx[None, :])
    C = jnp.cos(ang)
    S = jnp.sin(ang)

    const = pl.BlockSpec(None, lambda b, n: (0, 0))
    out = pl.pallas_call(
        _ac_kernel,
        grid=(B, N),
        in_specs=[
            pl.BlockSpec((1, 1, TP, PATCH * F), lambda b, n: (b, n, 0, 0)),
            pl.BlockSpec((PATCH * F, H), lambda b, n: (0, 0)),
            pl.BlockSpec((PATCH * F, H), lambda b, n: (0, 0)),
            pl.BlockSpec((PATCH * F, F), lambda b, n: (0, 0)),
            pl.BlockSpec((F, 1), lambda b, n: (0, 0)),
            pl.BlockSpec((TP, TP), lambda b, n: (0, 0)),
            pl.BlockSpec((TP, TP), lambda b, n: (0, 0)),
        ],
        out_specs=pl.BlockSpec((1, 1, F, T), lambda b, n: (b, n, 0, 0)),
        out_shape=jax.ShapeDtypeStruct((B, N, F, T), jnp.float32),
        compiler_params=pltpu.CompilerParams(
            dimension_semantics=("parallel", "parallel"),
        ),
        interpret=interpret,
    )(xr, Wq2, Wk2, Wvblk, Wsum, C, S)
    return out
